# Initial kernel scaffold; baseline (speedup 1.0000x reference)
import jax
import jax.numpy as jnp
from jax.experimental import pallas as pl


def kernel(x, edge_index, W):
    raise NotImplementedError("write your pallas kernel here")


# speedup vs baseline: 14.5287x; 14.5287x over previous
import functools

import jax
import jax.numpy as jnp
from jax import lax
from jax.experimental import pallas as pl
from jax.experimental.pallas import tpu as pltpu
from jax.experimental.pallas import tpu_sc as plsc

_NC = 2
_NS = 16
_NW = _NC * _NS
_K = 128
_D = 128


def _sc_mesh():
    return plsc.VectorSubcoreMesh(
        core_axis_name="c", subcore_axis_name="s",
        num_cores=_NC, num_subcores=_NS)


def _sc_degree(row_pad, n_pad, e_pad):
    ew = e_pad // _NW
    nt = n_pad // _NS
    nchunks = ew // _K

    @functools.partial(
        pl.kernel,
        out_type=jax.ShapeDtypeStruct((_NC, n_pad), jnp.float32),
        mesh=_sc_mesh(),
        scratch_types=[
            pltpu.VMEM((_K,), jnp.int32),
            pltpu.VMEM((_K,), jnp.float32),
            pltpu.VMEM((nt,), jnp.float32),
            pltpu.VMEM_SHARED((n_pad,), jnp.float32),
        ],
    )
    def deg_kernel(row_hbm, cnt_hbm, idx_v, ones_v, z_v, cnt_sp):
        c = lax.axis_index("c")
        s = lax.axis_index("s")
        wid = c * _NS + s

        def fill_ones(i, _):
            ones_v[pl.ds(i * 16, 16)] = jnp.full((16,), 1.0, jnp.float32)
            return 0
        lax.fori_loop(0, _K // 16, fill_ones, 0)

        def fill_zero(i, _):
            z_v[pl.ds(i * 16, 16)] = jnp.zeros((16,), jnp.float32)
            return 0
        lax.fori_loop(0, nt // 16, fill_zero, 0)

        pltpu.sync_copy(z_v, cnt_sp.at[pl.ds(s * nt, nt)])
        plsc.subcore_barrier()

        def body(t, _):
            base = wid * ew + t * _K
            pltpu.sync_copy(row_hbm.at[pl.ds(base, _K)], idx_v)
            pltpu.sync_copy(ones_v, cnt_sp.at[idx_v], add=True)
            return 0
        lax.fori_loop(0, nchunks, body, 0)

        plsc.subcore_barrier()
        pltpu.sync_copy(cnt_sp.at[pl.ds(s * nt, nt)],
                        cnt_hbm.at[c, pl.ds(s * nt, nt)])

    return deg_kernel(row_pad)


def _sc_aggregate(y, row_pad, col_pad, n_pad, e_pad):
    ew = e_pad // _NW
    nt = n_pad // _NS
    nchunks = ew // _K

    @functools.partial(
        pl.kernel,
        out_type=jax.ShapeDtypeStruct((_NC, n_pad, _D), jnp.float32),
        mesh=_sc_mesh(),
        scratch_types=[
            pltpu.VMEM((_K,), jnp.int32),
            pltpu.VMEM((_K,), jnp.int32),
            pltpu.VMEM((_K, _D), jnp.float32),
            pltpu.VMEM_SHARED((n_pad, _D), jnp.float32),
            pltpu.SemaphoreType.DMA,
        ],
    )
    def agg_kernel(y_hbm, row_hbm, col_hbm, out_hbm, cbuf, rbuf, rows_v, acc_sp, sem):
        c = lax.axis_index("c")
        s = lax.axis_index("s")
        wid = c * _NS + s

        def zrow(r, _):
            def zcol(l, _):
                rows_v[r, pl.ds(l * 16, 16)] = jnp.zeros((16,), jnp.float32)
                return 0
            lax.fori_loop(0, _D // 16, zcol, 0)
            return 0
        lax.fori_loop(0, _K, zrow, 0)

        def zacc(b, _):
            pltpu.sync_copy(rows_v, acc_sp.at[pl.ds(s * nt + b * _K, _K)])
            return 0
        lax.fori_loop(0, nt // _K, zacc, 0)
        plsc.subcore_barrier()

        def body(t, _):
            base = wid * ew + t * _K
            pltpu.sync_copy(col_hbm.at[pl.ds(base, _K)], cbuf)
            pltpu.sync_copy(row_hbm.at[pl.ds(base, _K)], rbuf)
            pltpu.async_copy(y_hbm.at[cbuf], rows_v, sem).wait()
            pltpu.sync_copy(rows_v, acc_sp.at[rbuf], add=True)
            return 0
        lax.fori_loop(0, nchunks, body, 0)

        plsc.subcore_barrier()
        pltpu.sync_copy(acc_sp.at[pl.ds(s * nt, nt)],
                        out_hbm.at[c, pl.ds(s * nt, nt)])

    return agg_kernel(y, row_pad, col_pad)


def _tc_transform(x_pad, W, cnt3):
    n_pad = x_pad.shape[0]
    blk = 256

    def body(x_ref, w_ref, cnt_ref, y_ref):
        xw = jnp.dot(x_ref[...], w_ref[...], preferred_element_type=jnp.float32)
        cnt = cnt_ref[...]
        dinv = lax.rsqrt(cnt[0] + cnt[1] + 2.0)
        y_ref[...] = xw * dinv

    return pl.pallas_call(
        body,
        grid=(n_pad // blk,),
        in_specs=[
            pl.BlockSpec((blk, _D), lambda i: (i, 0)),
            pl.BlockSpec((_D, _D), lambda i: (0, 0)),
            pl.BlockSpec((_NC, blk, 1), lambda i: (0, i, 0)),
        ],
        out_specs=pl.BlockSpec((blk, _D), lambda i: (i, 0)),
        out_shape=jax.ShapeDtypeStruct((n_pad, _D), jnp.float32),
    )(x_pad, W, cnt3)


def _tc_finalize(S, y, cnt3):
    n_pad = y.shape[0]
    blk = 256

    def body(s_ref, y_ref, cnt_ref, o_ref):
        cnt = cnt_ref[...]
        dinv = lax.rsqrt(cnt[0] + cnt[1] + 2.0)
        acc = s_ref[0] + s_ref[1] + 2.0 * y_ref[...]
        o_ref[...] = jnp.maximum(acc * dinv, 0.0)

    return pl.pallas_call(
        body,
        grid=(n_pad // blk,),
        in_specs=[
            pl.BlockSpec((_NC, blk, _D), lambda i: (0, i, 0)),
            pl.BlockSpec((blk, _D), lambda i: (i, 0)),
            pl.BlockSpec((_NC, blk, 1), lambda i: (0, i, 0)),
        ],
        out_specs=pl.BlockSpec((blk, _D), lambda i: (i, 0)),
        out_shape=jax.ShapeDtypeStruct((n_pad, _D), jnp.float32),
    )(S, y, cnt3)


def kernel(x, edge_index, W):
    n, d_in = x.shape
    e = edge_index.shape[1]

    n_pad = -(-n // (_NS * _K)) * (_NS * _K)
    e_pad = -(-e // (_NW * _K)) * (_NW * _K)

    row = edge_index[0]
    col = edge_index[1]
    pad_i = jnp.full((e_pad - e,), n_pad - 1, dtype=jnp.int32)
    row_p = jnp.concatenate([row, pad_i])
    col_p = jnp.concatenate([col, pad_i])
    x_p = jnp.pad(x, ((0, n_pad - n), (0, 0)))

    cnt = _sc_degree(row_p, n_pad, e_pad)
    cnt3 = cnt[:, :, None]
    y = _tc_transform(x_p, W, cnt3)
    S = _sc_aggregate(y, row_p, col_p, n_pad, e_pad)
    out = _tc_finalize(S, y, cnt3)
    return out[:n]
